# Initial kernel scaffold; baseline (speedup 1.0000x reference)
import jax
import jax.numpy as jnp
from jax.experimental import pallas as pl


def kernel(x_room, edge_h_index, edge_h_weight, edge_v_index, edge_v_weight, W_h, b_h, W_v, b_v, ln_weight, ln_bias):
    raise NotImplementedError("write your pallas kernel here")


# speedup vs baseline: 325.6446x; 325.6446x over previous
import functools

import jax
import jax.numpy as jnp
from jax import lax
from jax.experimental import pallas as pl
from jax.experimental.pallas import tpu as pltpu
from jax.experimental.pallas import tpu_sc as plsc

N = 10000
E = 80000
C = 16
TM = 16
NS = 16
PW = 125
PIECES = E // PW
PPT = PIECES // NS


def _sc_degree_body(eh_idx, eh_w, ev_idx, ev_w, zeros_hbm, out,
                    idx_v, w_v, acc):
    core = lax.axis_index("c")
    sub = lax.axis_index("s")

    @pl.when(sub == 0)
    def _zero():
        pltpu.sync_copy(zeros_hbm, acc)

    plsc.subcore_barrier()

    def _accumulate(idx3, w2):
        pltpu.sync_copy(idx3.at[0, pl.ds(sub * PPT, PPT)], idx_v.at[0])
        pltpu.sync_copy(idx3.at[1, pl.ds(sub * PPT, PPT)], idx_v.at[1])
        pltpu.sync_copy(w2.at[pl.ds(sub * PPT, PPT)], w_v)

        def body(j, carry):
            pltpu.sync_copy(w_v.at[j], acc.at[idx_v.at[0, j]], add=True)
            pltpu.sync_copy(w_v.at[j], acc.at[idx_v.at[1, j]], add=True)
            return carry

        lax.fori_loop(0, PPT, body, 0)

    @pl.when(core == 0)
    def _h():
        _accumulate(eh_idx, eh_w)

    @pl.when(core == 1)
    def _v():
        _accumulate(ev_idx, ev_w)

    plsc.subcore_barrier()

    @pl.when(sub == 0)
    def _out():
        pltpu.sync_copy(acc, out.at[core])


def _sc_degree(eh_idx, eh_w, ev_idx, ev_w):
    mesh = plsc.VectorSubcoreMesh(core_axis_name="c", subcore_axis_name="s")
    zeros = jnp.zeros((N,), jnp.float32)
    k = pl.kernel(
        _sc_degree_body,
        out_type=jax.ShapeDtypeStruct((2, N), jnp.float32),
        mesh=mesh,
        scratch_types=[
            pltpu.VMEM((2, PPT, PW), jnp.int32),
            pltpu.VMEM((PPT, PW), jnp.float32),
            pltpu.VMEM_SHARED((N,), jnp.float32),
        ],
    )
    return k(eh_idx.reshape(2, PIECES, PW), eh_w.reshape(PIECES, PW),
             ev_idx.reshape(2, PIECES, PW), ev_w.reshape(PIECES, PW),
             zeros)


def _tc_dense_body(x_ref, sw_ref, wh_ref, wv_ref, bh_ref, bv_ref,
                   lnw_ref, lnb_ref, out_ref):
    f32 = jnp.float32
    wh_t = wh_ref[...]
    wv_t = wv_ref[...]
    sw_h = sw_ref[0:1, :]
    sw_v = sw_ref[1:2, :]
    deg_h = 1.0 + sw_h
    deg_v = 1.0 + sw_v
    a_h = sw_h * lax.rsqrt(deg_h)
    a_v = sw_v * lax.rsqrt(deg_v)
    c_h = 1.0 / deg_h
    c_v = 1.0 / deg_v
    bias = bh_ref[...] + bv_ref[...]
    lnw = lnw_ref[...]
    lnb = lnb_ref[...]

    def norm(o):
        o = jnp.maximum(o, 0.0)
        mu = jnp.mean(o, axis=0, keepdims=True)
        d = o - mu
        var = jnp.mean(d * d, axis=0, keepdims=True)
        return d * lax.rsqrt(var + 1e-5) * lnw + lnb

    for t in range(TM):
        x0 = x_ref[0, :, t, :]
        x1 = x_ref[1, :, t, :]
        ph0 = jnp.dot(wh_t, x0, preferred_element_type=f32)
        pv0 = jnp.dot(wv_t, x0, preferred_element_type=f32)
        ph1 = jnp.dot(wh_t, x1, preferred_element_type=f32)
        pv1 = jnp.dot(wv_t, x1, preferred_element_type=f32)
        o0 = ph0 + pv0 + bias
        o1 = a_h * ph0 + c_h * ph1 + a_v * pv0 + c_v * pv1 + bias
        out_ref[0, :, t, :] = norm(o0)
        out_ref[1, :, t, :] = norm(o1)


def _tc_dense(x_room, sumw, W_h, b_h, W_v, b_v, ln_weight, ln_bias):
    return pl.pallas_call(
        _tc_dense_body,
        out_shape=jax.ShapeDtypeStruct((2, C, TM, N), jnp.float32),
    )(x_room[:, :, :TM, :], sumw, W_h.T, W_v.T,
      b_h.reshape(C, 1), b_v.reshape(C, 1),
      ln_weight.reshape(C, 1), ln_bias.reshape(C, 1))


def kernel(x_room, edge_h_index, edge_h_weight, edge_v_index, edge_v_weight,
           W_h, b_h, W_v, b_v, ln_weight, ln_bias):
    sumw = _sc_degree(edge_h_index, edge_h_weight,
                      edge_v_index, edge_v_weight)
    return _tc_dense(x_room, sumw, W_h, b_h, W_v, b_v, ln_weight, ln_bias)


# speedup vs baseline: 407.8671x; 1.2525x over previous
import functools

import jax
import jax.numpy as jnp
from jax import lax
from jax.experimental import pallas as pl
from jax.experimental.pallas import tpu as pltpu
from jax.experimental.pallas import tpu_sc as plsc

N = 10000
E = 80000
C = 16
TM = 16
NS = 16
PW = 125
PIECES = E // PW
PPT = PIECES // NS


def _sc_degree_body(eh_idx, eh_w, ev_idx, ev_w, zeros_hbm, out,
                    idx_v, w_v, acc, sem):
    core = lax.axis_index("c")
    sub = lax.axis_index("s")

    @pl.when(sub == 0)
    def _zero():
        pltpu.sync_copy(zeros_hbm, acc)

    plsc.subcore_barrier()

    def _accumulate(idx3, w2):
        pltpu.sync_copy(idx3.at[0, pl.ds(sub * PPT, PPT)], idx_v.at[0])
        pltpu.sync_copy(idx3.at[1, pl.ds(sub * PPT, PPT)], idx_v.at[1])
        pltpu.sync_copy(w2.at[pl.ds(sub * PPT, PPT)], w_v)

        descs = []
        for j in range(PPT):
            descs.append(pltpu.async_copy(
                w_v.at[j], acc.at[idx_v.at[0, j]], sem, add=True))
            descs.append(pltpu.async_copy(
                w_v.at[j], acc.at[idx_v.at[1, j]], sem, add=True))
        for d in descs:
            d.wait()

    @pl.when(core == 0)
    def _h():
        _accumulate(eh_idx, eh_w)

    @pl.when(core == 1)
    def _v():
        _accumulate(ev_idx, ev_w)

    plsc.subcore_barrier()

    @pl.when(sub == 0)
    def _out():
        pltpu.sync_copy(acc, out.at[core])


def _sc_degree(eh_idx, eh_w, ev_idx, ev_w):
    mesh = plsc.VectorSubcoreMesh(core_axis_name="c", subcore_axis_name="s")
    zeros = jnp.zeros((N,), jnp.float32)
    k = pl.kernel(
        _sc_degree_body,
        out_type=jax.ShapeDtypeStruct((2, N), jnp.float32),
        mesh=mesh,
        scratch_types=[
            pltpu.VMEM((2, PPT, PW), jnp.int32),
            pltpu.VMEM((PPT, PW), jnp.float32),
            pltpu.VMEM_SHARED((N,), jnp.float32),
            pltpu.SemaphoreType.DMA,
        ],
    )
    return k(eh_idx.reshape(2, PIECES, PW), eh_w.reshape(PIECES, PW),
             ev_idx.reshape(2, PIECES, PW), ev_w.reshape(PIECES, PW),
             zeros)


def _tc_dense_body(x_ref, sw_ref, wh_ref, wv_ref, bh_ref, bv_ref,
                   lnw_ref, lnb_ref, out_ref):
    f32 = jnp.float32
    wh_t = wh_ref[...]
    wv_t = wv_ref[...]
    sw_h = sw_ref[0:1, :]
    sw_v = sw_ref[1:2, :]
    deg_h = 1.0 + sw_h
    deg_v = 1.0 + sw_v
    a_h = sw_h * lax.rsqrt(deg_h)
    a_v = sw_v * lax.rsqrt(deg_v)
    c_h = 1.0 / deg_h
    c_v = 1.0 / deg_v
    bias = bh_ref[...] + bv_ref[...]
    lnw = lnw_ref[...]
    lnb = lnb_ref[...]

    def norm(o):
        o = jnp.maximum(o, 0.0)
        mu = jnp.mean(o, axis=0, keepdims=True)
        d = o - mu
        var = jnp.mean(d * d, axis=0, keepdims=True)
        return d * lax.rsqrt(var + 1e-5) * lnw + lnb

    for t in range(8):
        x0 = x_ref[0, :, t, :]
        x1 = x_ref[1, :, t, :]
        ph0 = jnp.dot(wh_t, x0, preferred_element_type=f32)
        pv0 = jnp.dot(wv_t, x0, preferred_element_type=f32)
        ph1 = jnp.dot(wh_t, x1, preferred_element_type=f32)
        pv1 = jnp.dot(wv_t, x1, preferred_element_type=f32)
        o0 = ph0 + pv0 + bias
        o1 = a_h * ph0 + c_h * ph1 + a_v * pv0 + c_v * pv1 + bias
        out_ref[0, :, t, :] = norm(o0)
        out_ref[1, :, t, :] = norm(o1)


def _tc_dense(x_room, sumw, W_h, b_h, W_v, b_v, ln_weight, ln_bias):
    T = x_room.shape[2]
    return pl.pallas_call(
        _tc_dense_body,
        grid=(2,),
        in_specs=[
            pl.BlockSpec((2, C, 8, N), lambda i: (0, 0, i, 0)),
            pl.BlockSpec((2, N), lambda i: (0, 0)),
            pl.BlockSpec((C, C), lambda i: (0, 0)),
            pl.BlockSpec((C, C), lambda i: (0, 0)),
            pl.BlockSpec((C, 1), lambda i: (0, 0)),
            pl.BlockSpec((C, 1), lambda i: (0, 0)),
            pl.BlockSpec((C, 1), lambda i: (0, 0)),
            pl.BlockSpec((C, 1), lambda i: (0, 0)),
        ],
        out_specs=pl.BlockSpec((2, C, 8, N), lambda i: (0, 0, i, 0)),
        out_shape=jax.ShapeDtypeStruct((2, C, TM, N), jnp.float32),
    )(x_room, sumw, W_h.T, W_v.T,
      b_h.reshape(C, 1), b_v.reshape(C, 1),
      ln_weight.reshape(C, 1), ln_bias.reshape(C, 1))


def kernel(x_room, edge_h_index, edge_h_weight, edge_v_index, edge_v_weight,
           W_h, b_h, W_v, b_v, ln_weight, ln_bias):
    sumw = _sc_degree(edge_h_index, edge_h_weight,
                      edge_v_index, edge_v_weight)
    return _tc_dense(x_room, sumw, W_h, b_h, W_v, b_v, ln_weight, ln_bias)
